# baseline (speedup 1.0000x reference)
import functools

import jax
import jax.numpy as jnp
from jax import lax
from jax.experimental import pallas as pl
from jax.experimental.pallas import tpu as pltpu
from jax.experimental.pallas import tpu_sc as plsc

D = 64
DH = D // 2
LANES = 16
KG = D // LANES
N_WORKERS = 32
CHUNK = 8


def _make_phase(n_tokens, final):
    per_w = n_tokens // N_WORKERS
    n_chunks = per_w // CHUNK
    mesh = plsc.VectorSubcoreMesh(core_axis_name="c", subcore_axis_name="s")

    @functools.partial(
        pl.kernel,
        mesh=mesh,
        out_type=jax.ShapeDtypeStruct((n_tokens * D,), jnp.float32),
        scratch_types=[
            pltpu.VMEM((CHUNK,), jnp.int32),
            pltpu.VMEM((CHUNK,), jnp.int32),
            pltpu.VMEM((CHUNK * DH,), jnp.float32),
            pltpu.VMEM((CHUNK * DH,), jnp.float32),
            pltpu.VMEM((CHUNK, DH * D), jnp.float32),
            pltpu.VMEM((CHUNK, DH * D), jnp.float32),
            pltpu.VMEM((CHUNK, 2 * D), jnp.float32),
            pltpu.VMEM((CHUNK, 2 * D), jnp.float32),
            pltpu.VMEM((CHUNK * D,), jnp.float32),
            pltpu.VMEM((CHUNK * D,), jnp.float32),
            pltpu.VMEM((CHUNK * D,), jnp.float32),
            pltpu.SemaphoreType.DMA,
            pltpu.SemaphoreType.DMA,
        ],
    )
    def k(x_hbm, u_hbm, t_hbm, b_hbm, p_hbm, out_hbm,
          idx0, idx1, x0, x1v, t0, t1, b0, b1, p0, p1, o_v, sem0, sem1):
        wid = lax.axis_index("s") * 2 + lax.axis_index("c")
        base = wid * per_w
        slots = (
            (idx0, x0, t0, b0, p0, sem0),
            (idx1, x1v, t1, b1, p1, sem1),
        )

        def fire(g, slot):
            idx_v, x_v, t_v, b_v, p_v, sem = slot
            start = base + g * CHUNK
            pltpu.sync_copy(u_hbm.at[pl.ds(start, CHUNK)], idx_v)
            pltpu.sync_copy(x_hbm.at[pl.ds(start * DH, CHUNK * DH)], x_v)
            if final:
                pltpu.sync_copy(p_hbm.at[pl.ds(start * D, CHUNK * D)], p_v)
                pltpu.async_copy(b_hbm.at[idx_v], b_v, sem)
            pltpu.async_copy(t_hbm.at[idx_v], t_v, sem)

        def compute(g, slot):
            idx_v, x_v, t_v, b_v, p_v, sem = slot
            pltpu.make_async_copy(t_hbm.at[idx_v], t_v, sem).wait()
            if final:
                pltpu.make_async_copy(b_hbm.at[idx_v], b_v, sem).wait()
            for t in range(CHUNK):
                if final:
                    acc_a = tuple(
                        b_v[t, pl.ds(kg * LANES, LANES)]
                        + p_v[pl.ds(t * D + kg * LANES, LANES)]
                        for kg in range(KG)
                    )
                else:
                    acc_a = tuple(
                        jnp.zeros((LANES,), jnp.float32) for _ in range(KG)
                    )
                acc_b = tuple(
                    jnp.zeros((LANES,), jnp.float32) for _ in range(KG)
                )

                def d_body(dg, accs, t=t):
                    acc_a, acc_b = accs
                    xv = x_v[pl.ds(t * DH + dg * LANES, LANES)]
                    for j in range(0, LANES, 2):
                        xb = jnp.full((LANES,), xv[j], jnp.float32)
                        row = (dg * LANES + j) * D
                        acc_a = tuple(
                            acc + xb * t_v[t, pl.ds(row + kg * LANES, LANES)]
                            for kg, acc in enumerate(acc_a)
                        )
                        xb2 = jnp.full((LANES,), xv[j + 1], jnp.float32)
                        row2 = row + D
                        acc_b = tuple(
                            acc + xb2 * t_v[t, pl.ds(row2 + kg * LANES, LANES)]
                            for kg, acc in enumerate(acc_b)
                        )
                    return acc_a, acc_b

                acc_a, acc_b = lax.fori_loop(
                    0, DH // LANES, d_body, (acc_a, acc_b), unroll=2)
                for kg in range(KG):
                    y = acc_a[kg] + acc_b[kg]
                    if final:
                        e = jnp.exp(y * 2.0)
                        y = 1.0 - 2.0 / (e + 1.0)
                    o_v[pl.ds(t * D + kg * LANES, LANES)] = y
            start = base + g * CHUNK
            pltpu.sync_copy(o_v, out_hbm.at[pl.ds(start * D, CHUNK * D)])

        fire(0, slots[0])

        def pair_body(p, carry):
            for s in range(2):
                g = p * 2 + s

                @pl.when(g + 1 < n_chunks)
                def _():
                    fire(g + 1, slots[1 - s])

                compute(g, slots[s])
            return carry

        lax.fori_loop(0, n_chunks // 2, pair_body, 0)

    return k


@functools.partial(jax.jit, static_argnames=("n_tokens",))
def _run(x1a, x1b, urls1, trans2a, trans2b, biasp, n_tokens):
    ka = _make_phase(n_tokens, final=False)
    kb = _make_phase(n_tokens, final=True)
    dummy_b = jnp.zeros((1, 2 * D), jnp.float32)
    dummy_p = jnp.zeros((n_tokens * D,), jnp.float32)
    partial = ka(x1a, urls1, trans2a, dummy_b, dummy_p)
    return kb(x1b, urls1, trans2b, biasp, partial)


def kernel(input, urls, trans, bias):
    B, S, d = input.shape
    n_tokens = B * S
    x1a = input[:, :, :DH].reshape(n_tokens * DH)
    x1b = input[:, :, DH:].reshape(n_tokens * DH)
    urls1 = urls.reshape(n_tokens).astype(jnp.int32)
    trans2a = trans[:, :DH, :].reshape(trans.shape[0], DH * d)
    trans2b = trans[:, DH:, :].reshape(trans.shape[0], DH * d)
    biasp = jnp.pad(bias, ((0, 0), (0, d)))
    out = _run(x1a, x1b, urls1, trans2a, trans2b, biasp, n_tokens)
    return out.reshape(input.shape)


# speedup vs baseline: 1.6881x; 1.6881x over previous
import functools

import jax
import jax.numpy as jnp
from jax import lax
from jax.experimental import pallas as pl
from jax.experimental.pallas import tpu as pltpu
from jax.experimental.pallas import tpu_sc as plsc

D = 64
LANES = 16
KG = D // LANES
N_WORKERS = 32
CHUNK = 8


@functools.partial(jax.jit, static_argnames=("n_tokens",))
def _run(x1, urls1, trans2, biasp, n_tokens):
    per_w = n_tokens // N_WORKERS
    n_chunks = per_w // CHUNK

    mesh = plsc.VectorSubcoreMesh(core_axis_name="c", subcore_axis_name="s")

    @functools.partial(
        pl.kernel,
        mesh=mesh,
        out_type=jax.ShapeDtypeStruct((n_tokens * D,), jnp.float32),
        scratch_types=[
            pltpu.VMEM((CHUNK,), jnp.int32),
            pltpu.VMEM((CHUNK,), jnp.int32),
            pltpu.VMEM((CHUNK * D,), jnp.float32),
            pltpu.VMEM((CHUNK * D,), jnp.float32),
            pltpu.VMEM((CHUNK, D * D), jnp.float32),
            pltpu.VMEM((CHUNK, D * D), jnp.float32),
            pltpu.VMEM((CHUNK, 2 * D), jnp.float32),
            pltpu.VMEM((CHUNK, 2 * D), jnp.float32),
            pltpu.VMEM((CHUNK * D,), jnp.float32),
            pltpu.SemaphoreType.DMA,
            pltpu.SemaphoreType.DMA,
        ],
    )
    def k(x_hbm, u_hbm, t_hbm, b_hbm, out_hbm,
          idx0, idx1, x0, x1v, t0, t1, b0, b1, o_v, sem0, sem1):
        wid = lax.axis_index("s") * 2 + lax.axis_index("c")
        base = wid * per_w
        slots = ((idx0, x0, t0, b0, sem0), (idx1, x1v, t1, b1, sem1))

        def fire(g, slot):
            idx_v, x_v, t_v, b_v, sem = slot
            start = base + g * CHUNK
            pltpu.sync_copy(u_hbm.at[pl.ds(start, CHUNK)], idx_v)
            pltpu.sync_copy(x_hbm.at[pl.ds(start * D, CHUNK * D)], x_v)
            pltpu.async_copy(t_hbm.at[idx_v], t_v, sem)
            pltpu.async_copy(b_hbm.at[idx_v], b_v, sem)

        def compute(g, slot):
            idx_v, x_v, t_v, b_v, sem = slot
            pltpu.make_async_copy(t_hbm.at[idx_v], t_v, sem).wait()
            pltpu.make_async_copy(b_hbm.at[idx_v], b_v, sem).wait()
            for t in range(CHUNK):
                acc_a = tuple(
                    b_v[t, pl.ds(kg * LANES, LANES)] for kg in range(KG)
                )
                acc_b = tuple(
                    jnp.zeros((LANES,), jnp.float32) for _ in range(KG)
                )

                def d_body(dg, accs, t=t):
                    acc_a, acc_b = accs
                    xv = x_v[pl.ds(t * D + dg * LANES, LANES)]
                    for j in range(0, LANES, 2):
                        xb = jnp.full((LANES,), xv[j], jnp.float32)
                        row = (dg * LANES + j) * D
                        acc_a = tuple(
                            acc + xb * t_v[t, pl.ds(row + kg * LANES, LANES)]
                            for kg, acc in enumerate(acc_a)
                        )
                        xb2 = jnp.full((LANES,), xv[j + 1], jnp.float32)
                        row2 = row + D
                        acc_b = tuple(
                            acc + xb2 * t_v[t, pl.ds(row2 + kg * LANES, LANES)]
                            for kg, acc in enumerate(acc_b)
                        )
                    return acc_a, acc_b

                acc_a, acc_b = lax.fori_loop(
                    0, KG, d_body, (acc_a, acc_b), unroll=2)
                for kg in range(KG):
                    e = jnp.exp((acc_a[kg] + acc_b[kg]) * 2.0)
                    o_v[pl.ds(t * D + kg * LANES, LANES)] = 1.0 - 2.0 / (e + 1.0)
            start = base + g * CHUNK
            pltpu.sync_copy(o_v, out_hbm.at[pl.ds(start * D, CHUNK * D)])

        fire(0, slots[0])

        def pair_body(p, carry):
            for s in range(2):
                g = p * 2 + s

                @pl.when(g + 1 < n_chunks)
                def _():
                    fire(g + 1, slots[1 - s])

                compute(g, slots[s])
            return carry

        lax.fori_loop(0, n_chunks // 2, pair_body, 0)

    return k(x1, urls1, trans2, biasp)


def kernel(input, urls, trans, bias):
    B, S, d = input.shape
    n_tokens = B * S
    x1 = input.reshape(n_tokens * d)
    urls1 = urls.reshape(n_tokens).astype(jnp.int32)
    trans2 = trans.reshape(trans.shape[0], d * d)
    biasp = jnp.pad(bias, ((0, 0), (0, d)))
    out = _run(x1, urls1, trans2, biasp, n_tokens)
    return out.reshape(input.shape)


# speedup vs baseline: 1.8017x; 1.0673x over previous
import functools

import jax
import jax.numpy as jnp
from jax import lax
from jax.experimental import pallas as pl
from jax.experimental.pallas import tpu as pltpu
from jax.experimental.pallas import tpu_sc as plsc

D = 64
LANES = 16
KG = D // LANES
N_WORKERS = 32
CHUNK = 8


@functools.partial(jax.jit, static_argnames=("n_tokens",))
def _run(x1, urls1, trans2, biasp, n_tokens):
    per_w = n_tokens // N_WORKERS
    n_chunks = per_w // CHUNK

    mesh = plsc.VectorSubcoreMesh(core_axis_name="c", subcore_axis_name="s")

    @functools.partial(
        pl.kernel,
        mesh=mesh,
        out_type=jax.ShapeDtypeStruct((n_tokens * D,), jnp.float32),
        scratch_types=[
            pltpu.VMEM((CHUNK,), jnp.int32),
            pltpu.VMEM((CHUNK,), jnp.int32),
            pltpu.VMEM((CHUNK * D,), jnp.float32),
            pltpu.VMEM((CHUNK * D,), jnp.float32),
            pltpu.VMEM((CHUNK, D * D), jnp.float32),
            pltpu.VMEM((CHUNK, D * D), jnp.float32),
            pltpu.VMEM((CHUNK, 2 * D), jnp.float32),
            pltpu.VMEM((CHUNK, 2 * D), jnp.float32),
            pltpu.VMEM((CHUNK * D,), jnp.float32),
            pltpu.VMEM((CHUNK * D,), jnp.float32),
            pltpu.SemaphoreType.DMA,
            pltpu.SemaphoreType.DMA,
            pltpu.SemaphoreType.DMA,
            pltpu.SemaphoreType.DMA,
        ],
    )
    def k(x_hbm, u_hbm, t_hbm, b_hbm, out_hbm,
          idx0, idx1, x0, x1v, t0, t1, b0, b1, o0, o1,
          sem0, sem1, semo0, semo1):
        wid = lax.axis_index("s") * 2 + lax.axis_index("c")
        base = wid * per_w
        slots = (
            (idx0, x0, t0, b0, o0, sem0, semo0),
            (idx1, x1v, t1, b1, o1, sem1, semo1),
        )

        def fire(g, slot):
            idx_v, x_v, t_v, b_v, o_v, sem, semo = slot
            start = base + g * CHUNK
            pltpu.sync_copy(u_hbm.at[pl.ds(start, CHUNK)], idx_v)
            pltpu.async_copy(x_hbm.at[pl.ds(start * D, CHUNK * D)], x_v, sem)
            pltpu.async_copy(t_hbm.at[idx_v], t_v, sem)
            pltpu.async_copy(b_hbm.at[idx_v], b_v, sem)

        def compute(g, slot):
            idx_v, x_v, t_v, b_v, o_v, sem, semo = slot
            start_prev = base + (g - 2) * CHUNK

            @pl.when(g >= 2)
            def _():
                pltpu.make_async_copy(
                    o_v, out_hbm.at[pl.ds(start_prev * D, CHUNK * D)],
                    semo).wait()

            start = base + g * CHUNK
            pltpu.make_async_copy(
                x_hbm.at[pl.ds(start * D, CHUNK * D)], x_v, sem).wait()
            pltpu.make_async_copy(t_hbm.at[idx_v], t_v, sem).wait()
            pltpu.make_async_copy(b_hbm.at[idx_v], b_v, sem).wait()
            for t in range(CHUNK):
                acc_a = tuple(
                    b_v[t, pl.ds(kg * LANES, LANES)] for kg in range(KG)
                )
                acc_b = tuple(
                    jnp.zeros((LANES,), jnp.float32) for _ in range(KG)
                )

                def d_body(dg, accs, t=t):
                    acc_a, acc_b = accs
                    xv = x_v[pl.ds(t * D + dg * LANES, LANES)]
                    for j in range(0, LANES, 2):
                        xb = jnp.full((LANES,), xv[j], jnp.float32)
                        row = (dg * LANES + j) * D
                        acc_a = tuple(
                            acc + xb * t_v[t, pl.ds(row + kg * LANES, LANES)]
                            for kg, acc in enumerate(acc_a)
                        )
                        xb2 = jnp.full((LANES,), xv[j + 1], jnp.float32)
                        row2 = row + D
                        acc_b = tuple(
                            acc + xb2 * t_v[t, pl.ds(row2 + kg * LANES, LANES)]
                            for kg, acc in enumerate(acc_b)
                        )
                    return acc_a, acc_b

                acc_a, acc_b = lax.fori_loop(
                    0, KG, d_body, (acc_a, acc_b), unroll=2)
                for kg in range(KG):
                    e = jnp.exp((acc_a[kg] + acc_b[kg]) * 2.0)
                    o_v[pl.ds(t * D + kg * LANES, LANES)] = 1.0 - 2.0 / (e + 1.0)
            pltpu.async_copy(
                o_v, out_hbm.at[pl.ds(start * D, CHUNK * D)], semo)

        fire(0, slots[0])

        def pair_body(p, carry):
            for s in range(2):
                g = p * 2 + s

                @pl.when(g + 1 < n_chunks)
                def _():
                    fire(g + 1, slots[1 - s])

                compute(g, slots[s])
            return carry

        lax.fori_loop(0, n_chunks // 2, pair_body, 0)

        for s in range(2):
            g_last = n_chunks - 2 + s
            idx_v, x_v, t_v, b_v, o_v, sem, semo = slots[g_last % 2]
            pltpu.make_async_copy(
                o_v, out_hbm.at[pl.ds((base + g_last * CHUNK) * D, CHUNK * D)],
                semo).wait()

    return k(x1, urls1, trans2, biasp)


def kernel(input, urls, trans, bias):
    B, S, d = input.shape
    n_tokens = B * S
    x1 = input.reshape(n_tokens * d)
    urls1 = urls.reshape(n_tokens).astype(jnp.int32)
    trans2 = trans.reshape(trans.shape[0], d * d)
    biasp = jnp.pad(bias, ((0, 0), (0, d)))
    out = _run(x1, urls1, trans2, biasp, n_tokens)
    return out.reshape(input.shape)


# speedup vs baseline: 1.8940x; 1.0512x over previous
import functools

import jax
import jax.numpy as jnp
from jax import lax
from jax.experimental import pallas as pl
from jax.experimental.pallas import tpu as pltpu
from jax.experimental.pallas import tpu_sc as plsc

D = 64
LANES = 16
KG = D // LANES
N_WORKERS = 32
CHUNK = 8


@functools.partial(jax.jit, static_argnames=("n_tokens",))
def _run(x1, urls1, trans2, biasp, n_tokens):
    per_w = n_tokens // N_WORKERS
    n_chunks = per_w // CHUNK

    mesh = plsc.VectorSubcoreMesh(core_axis_name="c", subcore_axis_name="s")

    @functools.partial(
        pl.kernel,
        mesh=mesh,
        out_type=jax.ShapeDtypeStruct((n_tokens * D,), jnp.float32),
        scratch_types=[
            pltpu.VMEM((per_w,), jnp.int32),
            pltpu.VMEM((per_w * D,), jnp.float32),
            pltpu.VMEM((CHUNK, D * D), jnp.float32),
            pltpu.VMEM((CHUNK, D * D), jnp.float32),
            pltpu.VMEM((CHUNK, 2 * D), jnp.float32),
            pltpu.VMEM((CHUNK, 2 * D), jnp.float32),
            pltpu.VMEM((CHUNK * D,), jnp.float32),
            pltpu.VMEM((CHUNK * D,), jnp.float32),
            pltpu.SemaphoreType.DMA,
            pltpu.SemaphoreType.DMA,
            pltpu.SemaphoreType.DMA,
            pltpu.SemaphoreType.DMA,
        ],
    )
    def k(x_hbm, u_hbm, t_hbm, b_hbm, out_hbm,
          u_all, x_all, t0, t1, b0, b1, o0, o1,
          sem0, sem1, semo0, semo1):
        wid = lax.axis_index("s") * 2 + lax.axis_index("c")
        base = wid * per_w
        slots = (
            (t0, b0, o0, sem0, semo0),
            (t1, b1, o1, sem1, semo1),
        )

        pltpu.sync_copy(u_hbm.at[pl.ds(base, per_w)], u_all)
        pltpu.sync_copy(x_hbm.at[pl.ds(base * D, per_w * D)], x_all)

        def fire(g, slot):
            t_v, b_v, o_v, sem, semo = slot
            idx_r = u_all.at[pl.ds(g * CHUNK, CHUNK)]
            pltpu.async_copy(t_hbm.at[idx_r], t_v, sem)
            pltpu.async_copy(b_hbm.at[idx_r], b_v, sem)

        def compute(g, slot):
            t_v, b_v, o_v, sem, semo = slot
            start_prev = base + (g - 2) * CHUNK

            @pl.when(g >= 2)
            def _():
                pltpu.make_async_copy(
                    o_v, out_hbm.at[pl.ds(start_prev * D, CHUNK * D)],
                    semo).wait()

            start = base + g * CHUNK
            idx_r = u_all.at[pl.ds(g * CHUNK, CHUNK)]
            pltpu.make_async_copy(t_hbm.at[idx_r], t_v, sem).wait()
            pltpu.make_async_copy(b_hbm.at[idx_r], b_v, sem).wait()
            for t in range(CHUNK):
                acc_a = tuple(
                    b_v[t, pl.ds(kg * LANES, LANES)] for kg in range(KG)
                )
                acc_b = tuple(
                    jnp.zeros((LANES,), jnp.float32) for _ in range(KG)
                )

                def d_body(dg, accs, t=t, g=g):
                    acc_a, acc_b = accs
                    xv = x_all[pl.ds((g * CHUNK + t) * D + dg * LANES, LANES)]
                    for j in range(0, LANES, 2):
                        xb = jnp.full((LANES,), xv[j], jnp.float32)
                        row = (dg * LANES + j) * D
                        acc_a = tuple(
                            acc + xb * t_v[t, pl.ds(row + kg * LANES, LANES)]
                            for kg, acc in enumerate(acc_a)
                        )
                        xb2 = jnp.full((LANES,), xv[j + 1], jnp.float32)
                        row2 = row + D
                        acc_b = tuple(
                            acc + xb2 * t_v[t, pl.ds(row2 + kg * LANES, LANES)]
                            for kg, acc in enumerate(acc_b)
                        )
                    return acc_a, acc_b

                acc_a, acc_b = lax.fori_loop(
                    0, KG, d_body, (acc_a, acc_b), unroll=2)
                for kg in range(KG):
                    e = jnp.exp((acc_a[kg] + acc_b[kg]) * 2.0)
                    o_v[pl.ds(t * D + kg * LANES, LANES)] = 1.0 - 2.0 / (e + 1.0)
            pltpu.async_copy(
                o_v, out_hbm.at[pl.ds(start * D, CHUNK * D)], semo)

        fire(0, slots[0])

        def pair_body(p, carry):
            for s in range(2):
                g = p * 2 + s

                @pl.when(g + 1 < n_chunks)
                def _():
                    fire(g + 1, slots[1 - s])

                compute(g, slots[s])
            return carry

        lax.fori_loop(0, n_chunks // 2, pair_body, 0)

        for s in range(2):
            g_last = n_chunks - 2 + s
            t_v, b_v, o_v, sem, semo = slots[g_last % 2]
            pltpu.make_async_copy(
                o_v, out_hbm.at[pl.ds((base + g_last * CHUNK) * D, CHUNK * D)],
                semo).wait()

    return k(x1, urls1, trans2, biasp)


def kernel(input, urls, trans, bias):
    B, S, d = input.shape
    n_tokens = B * S
    x1 = input.reshape(n_tokens * d)
    urls1 = urls.reshape(n_tokens).astype(jnp.int32)
    trans2 = trans.reshape(trans.shape[0], d * d)
    biasp = jnp.pad(bias, ((0, 0), (0, d)))
    out = _run(x1, urls1, trans2, biasp, n_tokens)
    return out.reshape(input.shape)
